# baseline (speedup 1.0000x reference)
import functools

import jax
import jax.numpy as jnp
from jax import lax
from jax.experimental import pallas as pl
from jax.experimental.pallas import tpu as pltpu
from jax.experimental.pallas import tpu_sc as plsc

N = 10000
E = 320000
D = 128

NC = 2
NS = 16
L = 16
NW = NC * NS
C = 96
T = 108
EPW = C * T
E_PAD = EPW * NW
N_PAD = 10240
ROWS_PER_SUB = N_PAD // NS
NB = 10


def _pq_body(h_ref, w_ref, d_ref, b_ref, p_ref, q_ref, hd_ref):
    r = lax.dot_general(
        w_ref[...], h_ref[...],
        dimension_numbers=(((1,), (1,)), ((), ())),
        preferred_element_type=jnp.float32,
    )
    p_ref[...] = r[0] + b_ref[0]
    q_ref[...] = r[1]
    hd_ref[...] = h_ref[...] * d_ref[...][:, None]


def _pq(h, w2, d, b_gate):
    return pl.pallas_call(
        _pq_body,
        out_shape=[jax.ShapeDtypeStruct((N,), jnp.float32),
                   jax.ShapeDtypeStruct((N,), jnp.float32),
                   jax.ShapeDtypeStruct((N, D), jnp.float32)],
        in_specs=[
            pl.BlockSpec(memory_space=pltpu.VMEM),
            pl.BlockSpec(memory_space=pltpu.VMEM),
            pl.BlockSpec(memory_space=pltpu.VMEM),
            pl.BlockSpec(memory_space=pltpu.SMEM),
        ],
        out_specs=[pl.BlockSpec(memory_space=pltpu.VMEM),
                   pl.BlockSpec(memory_space=pltpu.VMEM),
                   pl.BlockSpec(memory_space=pltpu.VMEM)],
    )(h, w2, d, b_gate)


_SC_MESH = plsc.VectorSubcoreMesh(core_axis_name="c", subcore_axis_name="s")


@functools.partial(
    pl.kernel,
    out_type=jax.ShapeDtypeStruct((NC, N_PAD, D), jnp.float32),
    mesh=_SC_MESH,
    scratch_types=[
        pltpu.VMEM((N,), jnp.float32),
        pltpu.VMEM((N,), jnp.float32),
        [pltpu.VMEM((2, C), jnp.int32)] * 3,
        [pltpu.VMEM((C, D), jnp.float32)] * 2,
        pltpu.VMEM((C,), jnp.float32),
        pltpu.VMEM_SHARED((N_PAD, D), jnp.float32),
        [pltpu.SemaphoreType.DMA] * 3,
        [pltpu.SemaphoreType.DMA] * 2,
        [pltpu.SemaphoreType.DMA] * 2,
    ],
    compiler_params=pltpu.CompilerParams(use_tc_tiling_on_sc=False,
                                         needs_layout_passes=False),
)
def _sc_edges(hd_hbm, idx_hbm, p_hbm, q_hbm, z0_hbm, zp_hbm,
              p_v, q_v, idx, rows, e_v, zacc, isem, gsem, ssem):
    cid = lax.axis_index("c")
    sid = lax.axis_index("s")
    wid = cid * NS + sid

    pltpu.sync_copy(p_hbm, p_v)
    pltpu.sync_copy(q_hbm, q_v)
    pltpu.sync_copy(z0_hbm.at[pl.ds(sid * ROWS_PER_SUB, ROWS_PER_SUB)],
                    zacc.at[pl.ds(sid * ROWS_PER_SUB, ROWS_PER_SUB)])
    plsc.subcore_barrier()

    cbase = wid * T

    pltpu.sync_copy(idx_hbm.at[cbase], idx[0])
    pltpu.async_copy(hd_hbm.at[idx[0].at[0]], rows[0], gsem[0])

    def group(gg, carry):
        for k in range(6):
            t = 6 * gg + k
            rb = k % 2
            ib = k % 3
            ibn = (k + 2) % 3

            @pl.when(t + 1 < T)
            def _():
                @pl.when(t >= 1)
                def _():
                    pltpu.make_async_copy(
                        rows[1 - rb], zacc.at[idx[(k + 2) % 3].at[1]],
                        ssem[1 - rb]).wait()
                pltpu.sync_copy(idx_hbm.at[cbase + t + 1], idx[(k + 1) % 3])
                pltpu.async_copy(hd_hbm.at[idx[(k + 1) % 3].at[0]],
                                 rows[1 - rb], gsem[1 - rb])

            ebase = (cbase + t) * C

            @plsc.parallel_loop(0, C // L, 1, unroll=2)
            def _(i):
                s16 = idx[ib][0, pl.ds(i * L, L)]
                d16 = idx[ib][1, pl.ds(i * L, L)]
                x = plsc.load_gather(p_v, [d16]) + plsc.load_gather(q_v, [s16])
                u = jnp.exp(jnp.abs(x) * -2.0)
                th = (1.0 - u) / (1.0 + u)
                th = jnp.where(x < 0.0, -th, th)
                gid = ebase + i * L + lax.iota(jnp.int32, L)
                e_v[pl.ds(i * L, L)] = jnp.where(gid < E, th, 0.0)

            pltpu.make_async_copy(hd_hbm.at[idx[ib].at[0]], rows[rb],
                                  gsem[rb]).wait()

            @plsc.parallel_loop(0, C, 1, unroll=2)
            def _(i):
                bc = plsc.load_gather(e_v, [jnp.full((L,), i, jnp.int32)])
                for j in range(D // L):
                    rows[rb][i, pl.ds(j * L, L)] = (
                        rows[rb][i, pl.ds(j * L, L)] * bc)

            pltpu.async_copy(rows[rb], zacc.at[idx[ib].at[1]], ssem[rb],
                             add=True)
        return carry

    lax.fori_loop(0, T // 6, group, 0)
    pltpu.make_async_copy(rows[0], zacc.at[idx[1].at[1]], ssem[0]).wait()
    pltpu.make_async_copy(rows[1], zacc.at[idx[2].at[1]], ssem[1]).wait()
    plsc.subcore_barrier()
    pltpu.sync_copy(zacc.at[pl.ds(sid * ROWS_PER_SUB, ROWS_PER_SUB)],
                    zp_hbm.at[cid, pl.ds(sid * ROWS_PER_SUB, ROWS_PER_SUB)])


def _add_body(zp_ref, d_ref, out_ref):
    out_ref[...] = (zp_ref[0] + zp_ref[1]) * d_ref[0, 0][:, None]


def _combine(zp, d2):
    zsum = pl.pallas_call(
        _add_body,
        grid=(NB,),
        out_shape=jax.ShapeDtypeStruct((N_PAD, D), jnp.float32),
        in_specs=[pl.BlockSpec((2, N_PAD // NB, D), lambda i: (0, i, 0)),
                  pl.BlockSpec((1, 1, N_PAD // NB), lambda i: (i, 0, 0))],
        out_specs=pl.BlockSpec((N_PAD // NB, D), lambda i: (i, 0)),
    )(zp, d2)
    return zsum[:N]


@jax.jit
def kernel(h, edge_index, d, W_gate, b_gate):
    src = edge_index[0]
    dst = edge_index[1]
    pad = E_PAD - E
    src_p = jnp.concatenate([src, jnp.zeros((pad,), jnp.int32)])
    dst_p = jnp.concatenate([dst, jnp.zeros((pad,), jnp.int32)])
    idx_c = jnp.stack([src_p.reshape(E_PAD // C, C),
                       dst_p.reshape(E_PAD // C, C)], axis=1)
    w2 = W_gate.reshape(2, D)
    p, q, hd = _pq(h, w2, d, b_gate)
    z0 = jnp.zeros((N_PAD, D), jnp.float32)
    zp = _sc_edges(hd, idx_c, p, q, z0)
    d2 = jnp.concatenate([d, jnp.zeros((N_PAD - N,), jnp.float32)]
                         ).reshape(NB, 1, N_PAD // NB)
    return _combine(zp, d2)
